# Initial kernel scaffold; baseline (speedup 1.0000x reference)
import jax
import jax.numpy as jnp
from jax.experimental import pallas as pl


def kernel(x, adj, W1, b1, W2, b2, W3, b3):
    raise NotImplementedError("write your pallas kernel here")


# speedup vs baseline: 4.7872x; 4.7872x over previous
import functools

import jax
import jax.numpy as jnp
from jax import lax
from jax.experimental import pallas as pl
from jax.experimental.pallas import tpu as pltpu
from jax.experimental.pallas import tpu_sc as plsc

N_NODES = 10000
N_EDGES = 320000
NS = 16
NC = 2
CHUNK = 80
ROWS_PER_SUB = 624
ROWS_TAIL = N_NODES - NS * ROWS_PER_SUB
D = 128


def _zero_acc(zeros, acc, s):
    row0 = pl.multiple_of(s * ROWS_PER_SUB, 8)
    pltpu.sync_copy(zeros.at[pl.ds(0, ROWS_PER_SUB)],
                    acc.at[pl.ds(row0, ROWS_PER_SUB)])

    @pl.when(s == NS - 1)
    def _():
        pltpu.sync_copy(zeros.at[pl.ds(0, ROWS_TAIL)],
                        acc.at[pl.ds(NS * ROWS_PER_SUB, ROWS_TAIL)])


def _copy_out(acc, out, s):
    row0 = pl.multiple_of(s * ROWS_PER_SUB, 8)
    pltpu.sync_copy(acc.at[pl.ds(row0, ROWS_PER_SUB)],
                    out.at[pl.ds(row0, ROWS_PER_SUB)])

    @pl.when(s == NS - 1)
    def _():
        pltpu.sync_copy(acc.at[pl.ds(NS * ROWS_PER_SUB, ROWS_TAIL)],
                        out.at[pl.ds(NS * ROWS_PER_SUB, ROWS_TAIL)])


def _edge_loop(h, src, dst, src_v, dst_v, rows_v, acc, sem, e_base, n_chunks):
    def body(i, carry):
        base = pl.multiple_of(e_base + i * CHUNK, 8)
        pltpu.sync_copy(src.at[pl.ds(base, CHUNK)], src_v.at[0])
        pltpu.sync_copy(dst.at[pl.ds(base, CHUNK)], dst_v.at[0])
        pltpu.async_copy(h.at[src_v.at[0]], rows_v, sem).wait()
        pltpu.sync_copy(rows_v, acc.at[dst_v.at[0]], add=True)
        return carry

    lax.fori_loop(0, n_chunks, body, 0)


_SC_SCRATCH = [
    pltpu.VMEM((1, CHUNK), jnp.int32),
    pltpu.VMEM((1, CHUNK), jnp.int32),
    pltpu.VMEM((CHUNK, D), jnp.float32),
    pltpu.VMEM_SHARED((N_NODES, D), jnp.float32),
    pltpu.SemaphoreType.DMA,
]

_mesh = plsc.VectorSubcoreMesh(core_axis_name="c", subcore_axis_name="s")


@functools.partial(
    pl.kernel,
    out_type=[
        jax.ShapeDtypeStruct((N_NODES, D), jnp.float32),
        jax.ShapeDtypeStruct((N_NODES, D), jnp.float32),
    ],
    mesh=_mesh,
    scratch_types=_SC_SCRATCH,
)
def _prop_edges(h, src, dst, zeros, p0, p1, src_v, dst_v, rows_v, acc, sem):
    c = lax.axis_index("c")
    s = lax.axis_index("s")
    _zero_acc(zeros, acc, s)
    plsc.subcore_barrier()

    per_sub = N_EDGES // (NC * NS)
    e_base = (c * NS + s) * per_sub
    _edge_loop(h, src, dst, src_v, dst_v, rows_v, acc, sem,
               e_base, per_sub // CHUNK)
    plsc.subcore_barrier()

    @pl.when(c == 0)
    def _():
        _copy_out(acc, p0, s)

    @pl.when(c == 1)
    def _():
        _copy_out(acc, p1, s)


@functools.partial(
    pl.kernel,
    out_type=[
        jax.ShapeDtypeStruct((N_NODES, D), jnp.float32),
        jax.ShapeDtypeStruct((N_NODES, D), jnp.float32),
    ],
    mesh=_mesh,
    scratch_types=_SC_SCRATCH,
)
def _prop_cols(h0, h1, src, dst, zeros, o0, o1, src_v, dst_v, rows_v, acc, sem):
    c = lax.axis_index("c")
    s = lax.axis_index("s")
    _zero_acc(zeros, acc, s)
    plsc.subcore_barrier()

    per_sub = N_EDGES // NS
    e_base = s * per_sub
    n_chunks = per_sub // CHUNK

    @pl.when(c == 0)
    def _():
        _edge_loop(h0, src, dst, src_v, dst_v, rows_v, acc, sem,
                   e_base, n_chunks)

    @pl.when(c == 1)
    def _():
        _edge_loop(h1, src, dst, src_v, dst_v, rows_v, acc, sem,
                   e_base, n_chunks)

    plsc.subcore_barrier()

    @pl.when(c == 0)
    def _():
        _copy_out(acc, o0, s)

    @pl.when(c == 1)
    def _():
        _copy_out(acc, o1, s)


_BLK = 1000
_GRID = N_NODES // _BLK


def _tc1_body(p0, p1, w, b, o0, o1):
    a = p0[...] + p1[...]
    h = jnp.dot(a, w[...], preferred_element_type=jnp.float32) + b[...]
    h = jnp.maximum(h, 0.0)
    o0[...] = h[:, 0:128]
    o1[...] = h[:, 128:256]


def _tc1(p0, p1, w1, b1):
    return pl.pallas_call(
        _tc1_body,
        grid=(_GRID,),
        in_specs=[
            pl.BlockSpec((_BLK, 128), lambda i: (i, 0)),
            pl.BlockSpec((_BLK, 128), lambda i: (i, 0)),
            pl.BlockSpec((128, 256), lambda i: (0, 0)),
            pl.BlockSpec((1, 256), lambda i: (0, 0)),
        ],
        out_specs=[
            pl.BlockSpec((_BLK, 128), lambda i: (i, 0)),
            pl.BlockSpec((_BLK, 128), lambda i: (i, 0)),
        ],
        out_shape=[
            jax.ShapeDtypeStruct((N_NODES, 128), jnp.float32),
            jax.ShapeDtypeStruct((N_NODES, 128), jnp.float32),
        ],
    )(p0, p1, w1, b1)


def _tc2_body(g0, g1, w2, b2, w3, t):
    h = (jnp.dot(g0[...], w2[0:128, :], preferred_element_type=jnp.float32)
         + jnp.dot(g1[...], w2[128:256, :], preferred_element_type=jnp.float32)
         + b2[...])
    h = jnp.maximum(h, 0.0)
    t[...] = jnp.dot(h, w3[...], preferred_element_type=jnp.float32)


def _tc2(g0, g1, w2, b2, w3):
    return pl.pallas_call(
        _tc2_body,
        grid=(_GRID,),
        in_specs=[
            pl.BlockSpec((_BLK, 128), lambda i: (i, 0)),
            pl.BlockSpec((_BLK, 128), lambda i: (i, 0)),
            pl.BlockSpec((256, 256), lambda i: (0, 0)),
            pl.BlockSpec((1, 256), lambda i: (0, 0)),
            pl.BlockSpec((256, 128), lambda i: (0, 0)),
        ],
        out_specs=pl.BlockSpec((_BLK, 128), lambda i: (i, 0)),
        out_shape=jax.ShapeDtypeStruct((N_NODES, 128), jnp.float32),
    )(g0, g1, w2, b2, w3)


def _tc3_body(q0, q1, b3, out):
    out[...] = q0[...] + q1[...] + b3[...]


def _tc3(q0, q1, b3):
    return pl.pallas_call(
        _tc3_body,
        grid=(_GRID,),
        in_specs=[
            pl.BlockSpec((_BLK, 128), lambda i: (i, 0)),
            pl.BlockSpec((_BLK, 128), lambda i: (i, 0)),
            pl.BlockSpec((1, 128), lambda i: (0, 0)),
        ],
        out_specs=pl.BlockSpec((_BLK, 128), lambda i: (i, 0)),
        out_shape=jax.ShapeDtypeStruct((N_NODES, 128), jnp.float32),
    )(q0, q1, b3)


def kernel(x, adj, W1, b1, W2, b2, W3, b3):
    src = adj[0].astype(jnp.int32)
    dst = adj[1].astype(jnp.int32)
    zeros = jnp.zeros((ROWS_PER_SUB, D), jnp.float32)

    p0, p1 = _prop_edges(x, src, dst, zeros)
    h0, h1 = _tc1(p0, p1, W1, b1.reshape(1, -1))

    g0, g1 = _prop_cols(h0, h1, src, dst, zeros)
    t = _tc2(g0, g1, W2, b2.reshape(1, -1), W3)

    q0, q1 = _prop_edges(t, src, dst, zeros)
    return _tc3(q0, q1, b3.reshape(1, -1))


# speedup vs baseline: 12.3395x; 2.5776x over previous
import functools

import jax
import jax.numpy as jnp
from jax import lax
from jax.experimental import pallas as pl
from jax.experimental.pallas import tpu as pltpu
from jax.experimental.pallas import tpu_sc as plsc

N_NODES = 10000
N_EDGES = 320000
NS = 16
NC = 2
CHUNK = 80
ROWS_PER_SUB = 624
ROWS_TAIL = N_NODES - NS * ROWS_PER_SUB
D = 128


def _zero_acc(zeros, acc, s):
    row0 = pl.multiple_of(s * ROWS_PER_SUB, 8)
    pltpu.sync_copy(zeros.at[pl.ds(0, ROWS_PER_SUB)],
                    acc.at[pl.ds(row0, ROWS_PER_SUB)])

    @pl.when(s == NS - 1)
    def _():
        pltpu.sync_copy(zeros.at[pl.ds(0, ROWS_TAIL)],
                        acc.at[pl.ds(NS * ROWS_PER_SUB, ROWS_TAIL)])


def _copy_out(acc, out, s):
    row0 = pl.multiple_of(s * ROWS_PER_SUB, 8)
    pltpu.sync_copy(acc.at[pl.ds(row0, ROWS_PER_SUB)],
                    out.at[pl.ds(row0, ROWS_PER_SUB)])

    @pl.when(s == NS - 1)
    def _():
        pltpu.sync_copy(acc.at[pl.ds(NS * ROWS_PER_SUB, ROWS_TAIL)],
                        out.at[pl.ds(NS * ROWS_PER_SUB, ROWS_TAIL)])


def _edge_loop(h, src_slab, dst_slab, rows, acc, sems, n_chunks):
    nbuf = len(rows)

    def _issue(i, b):
        pltpu.async_copy(h.at[src_slab.at[i]], rows[b], sems[b])

    def _wait(b):
        pltpu.make_async_copy(h.at[src_slab.at[0]], rows[b], sems[b]).wait()

    def _scatter(i, b):
        pltpu.sync_copy(rows[b], acc.at[dst_slab.at[i]], add=True)

    for b in range(nbuf - 1):
        _issue(b, b)

    n_main = n_chunks // nbuf * nbuf

    def body(p, carry):
        i0 = p * nbuf
        for b in range(nbuf):
            i = i0 + b
            _wait(b)

            @pl.when(i + nbuf - 1 < n_chunks)
            def _():
                _issue(i + nbuf - 1, (b + nbuf - 1) % nbuf)

            _scatter(i, b)
        return carry

    lax.fori_loop(0, n_main // nbuf, body, 0)

    for i in range(n_main, n_chunks):
        b = i % nbuf
        _wait(b)

        @pl.when(i + nbuf - 1 < n_chunks)
        def _():
            _issue(i + nbuf - 1, (b + nbuf - 1) % nbuf)

        _scatter(i, b)


SUPER = 25
_CHUNKS_E = N_EDGES // (NC * NS) // CHUNK
_CHUNKS_C = N_EDGES // NS // CHUNK
_SUPER_E = _CHUNKS_E // SUPER
_SUPER_C = _CHUNKS_C // SUPER

NBUF = 3

_SC_SCRATCH = (
    [pltpu.VMEM((1, 1, SUPER, CHUNK), jnp.int32)] * 2
    + [pltpu.VMEM((CHUNK, D), jnp.float32)] * NBUF
    + [pltpu.VMEM_SHARED((N_NODES, D), jnp.float32)]
    + [pltpu.SemaphoreType.DMA] * NBUF
)


_mesh = plsc.VectorSubcoreMesh(core_axis_name="c", subcore_axis_name="s")


@functools.partial(
    pl.kernel,
    out_type=[
        jax.ShapeDtypeStruct((N_NODES, D), jnp.float32),
        jax.ShapeDtypeStruct((N_NODES, D), jnp.float32),
    ],
    mesh=_mesh,
    scratch_types=_SC_SCRATCH,
)
def _prop_edges(h, src, dst, zeros, p0, p1,
                src_slab, dst_slab, *rest):
    rows, acc, sems = rest[:NBUF], rest[NBUF], rest[NBUF + 1:]
    c = lax.axis_index("c")
    s = lax.axis_index("s")
    w = c * NS + s
    _zero_acc(zeros, acc, s)
    plsc.subcore_barrier()

    def outer(j, carry):
        pltpu.sync_copy(src.at[pl.ds(w, 1), pl.ds(j, 1)], src_slab)
        pltpu.sync_copy(dst.at[pl.ds(w, 1), pl.ds(j, 1)], dst_slab)
        _edge_loop(h, src_slab.at[0, 0], dst_slab.at[0, 0], rows,
                   acc, sems, SUPER)
        return carry

    lax.fori_loop(0, _SUPER_E, outer, 0)
    plsc.subcore_barrier()

    @pl.when(c == 0)
    def _():
        _copy_out(acc, p0, s)

    @pl.when(c == 1)
    def _():
        _copy_out(acc, p1, s)


@functools.partial(
    pl.kernel,
    out_type=[
        jax.ShapeDtypeStruct((N_NODES, D), jnp.float32),
        jax.ShapeDtypeStruct((N_NODES, D), jnp.float32),
    ],
    mesh=_mesh,
    scratch_types=_SC_SCRATCH,
)
def _prop_cols(h0, h1, src, dst, zeros, o0, o1,
               src_slab, dst_slab, *rest):
    rows, acc, sems = rest[:NBUF], rest[NBUF], rest[NBUF + 1:]
    c = lax.axis_index("c")
    s = lax.axis_index("s")
    _zero_acc(zeros, acc, s)
    plsc.subcore_barrier()

    def _loop_for(h):
        def outer(j, carry):
            pltpu.sync_copy(src.at[pl.ds(s, 1), pl.ds(j, 1)], src_slab)
            pltpu.sync_copy(dst.at[pl.ds(s, 1), pl.ds(j, 1)], dst_slab)
            _edge_loop(h, src_slab.at[0, 0], dst_slab.at[0, 0], rows,
                       acc, sems, SUPER)
            return carry

        lax.fori_loop(0, _SUPER_C, outer, 0)

    @pl.when(c == 0)
    def _():
        _loop_for(h0)

    @pl.when(c == 1)
    def _():
        _loop_for(h1)

    plsc.subcore_barrier()

    @pl.when(c == 0)
    def _():
        _copy_out(acc, o0, s)

    @pl.when(c == 1)
    def _():
        _copy_out(acc, o1, s)


_BLK = 1000
_GRID = N_NODES // _BLK


def _tc1_body(p0, p1, w, b, o0, o1):
    a = p0[...] + p1[...]
    h = jnp.dot(a, w[...], preferred_element_type=jnp.float32) + b[...]
    h = jnp.maximum(h, 0.0)
    o0[...] = h[:, 0:128]
    o1[...] = h[:, 128:256]


def _tc1(p0, p1, w1, b1):
    return pl.pallas_call(
        _tc1_body,
        grid=(_GRID,),
        in_specs=[
            pl.BlockSpec((_BLK, 128), lambda i: (i, 0)),
            pl.BlockSpec((_BLK, 128), lambda i: (i, 0)),
            pl.BlockSpec((128, 256), lambda i: (0, 0)),
            pl.BlockSpec((1, 256), lambda i: (0, 0)),
        ],
        out_specs=[
            pl.BlockSpec((_BLK, 128), lambda i: (i, 0)),
            pl.BlockSpec((_BLK, 128), lambda i: (i, 0)),
        ],
        out_shape=[
            jax.ShapeDtypeStruct((N_NODES, 128), jnp.float32),
            jax.ShapeDtypeStruct((N_NODES, 128), jnp.float32),
        ],
    )(p0, p1, w1, b1)


def _tc2_body(g0, g1, w2, b2, w3, t):
    h = (jnp.dot(g0[...], w2[0:128, :], preferred_element_type=jnp.float32)
         + jnp.dot(g1[...], w2[128:256, :], preferred_element_type=jnp.float32)
         + b2[...])
    h = jnp.maximum(h, 0.0)
    t[...] = jnp.dot(h, w3[...], preferred_element_type=jnp.float32)


def _tc2(g0, g1, w2, b2, w3):
    return pl.pallas_call(
        _tc2_body,
        grid=(_GRID,),
        in_specs=[
            pl.BlockSpec((_BLK, 128), lambda i: (i, 0)),
            pl.BlockSpec((_BLK, 128), lambda i: (i, 0)),
            pl.BlockSpec((256, 256), lambda i: (0, 0)),
            pl.BlockSpec((1, 256), lambda i: (0, 0)),
            pl.BlockSpec((256, 128), lambda i: (0, 0)),
        ],
        out_specs=pl.BlockSpec((_BLK, 128), lambda i: (i, 0)),
        out_shape=jax.ShapeDtypeStruct((N_NODES, 128), jnp.float32),
    )(g0, g1, w2, b2, w3)


def _tc3_body(q0, q1, b3, out):
    out[...] = q0[...] + q1[...] + b3[...]


def _tc3(q0, q1, b3):
    return pl.pallas_call(
        _tc3_body,
        grid=(_GRID,),
        in_specs=[
            pl.BlockSpec((_BLK, 128), lambda i: (i, 0)),
            pl.BlockSpec((_BLK, 128), lambda i: (i, 0)),
            pl.BlockSpec((1, 128), lambda i: (0, 0)),
        ],
        out_specs=pl.BlockSpec((_BLK, 128), lambda i: (i, 0)),
        out_shape=jax.ShapeDtypeStruct((N_NODES, 128), jnp.float32),
    )(q0, q1, b3)


def kernel(x, adj, W1, b1, W2, b2, W3, b3):
    src = adj[0].astype(jnp.int32)
    dst = adj[1].astype(jnp.int32)
    src_e = src.reshape(NC * NS, _SUPER_E, SUPER, CHUNK)
    dst_e = dst.reshape(NC * NS, _SUPER_E, SUPER, CHUNK)
    src_c = src.reshape(NS, _SUPER_C, SUPER, CHUNK)
    dst_c = dst.reshape(NS, _SUPER_C, SUPER, CHUNK)
    zeros = jnp.zeros((ROWS_PER_SUB, D), jnp.float32)

    p0, p1 = _prop_edges(x, src_e, dst_e, zeros)
    h0, h1 = _tc1(p0, p1, W1, b1.reshape(1, -1))

    g0, g1 = _prop_cols(h0, h1, src_c, dst_c, zeros)
    t = _tc2(g0, g1, W2, b2.reshape(1, -1), W3)

    q0, q1 = _prop_edges(t, src_e, dst_e, zeros)
    return _tc3(q0, q1, b3.reshape(1, -1))


# speedup vs baseline: 12.6171x; 1.0225x over previous
import functools

import jax
import jax.numpy as jnp
from jax import lax
from jax.experimental import pallas as pl
from jax.experimental.pallas import tpu as pltpu
from jax.experimental.pallas import tpu_sc as plsc

N_NODES = 10000
N_EDGES = 320000
NS = 16
NC = 2
CHUNK = 80
ROWS_PER_SUB = 624
ROWS_TAIL = N_NODES - NS * ROWS_PER_SUB
D = 128


def _zero_acc(zeros, acc, s):
    row0 = pl.multiple_of(s * ROWS_PER_SUB, 8)
    pltpu.sync_copy(zeros.at[pl.ds(0, ROWS_PER_SUB)],
                    acc.at[pl.ds(row0, ROWS_PER_SUB)])

    @pl.when(s == NS - 1)
    def _():
        pltpu.sync_copy(zeros.at[pl.ds(0, ROWS_TAIL)],
                        acc.at[pl.ds(NS * ROWS_PER_SUB, ROWS_TAIL)])


def _copy_out(acc, out, s):
    row0 = pl.multiple_of(s * ROWS_PER_SUB, 8)
    pltpu.sync_copy(acc.at[pl.ds(row0, ROWS_PER_SUB)],
                    out.at[pl.ds(row0, ROWS_PER_SUB)])

    @pl.when(s == NS - 1)
    def _():
        pltpu.sync_copy(acc.at[pl.ds(NS * ROWS_PER_SUB, ROWS_TAIL)],
                        out.at[pl.ds(NS * ROWS_PER_SUB, ROWS_TAIL)])


def _edge_loop(h, src_slab, dst_slab, rows, acc, sems, n_chunks):
    nbuf = len(rows)

    def _issue(i, b):
        pltpu.async_copy(h.at[src_slab.at[i]], rows[b], sems[b])

    def _wait(b):
        pltpu.make_async_copy(h.at[src_slab.at[0]], rows[b], sems[b]).wait()

    def _scatter(i, b):
        pltpu.sync_copy(rows[b], acc.at[dst_slab.at[i]], add=True)

    for b in range(nbuf - 1):
        _issue(b, b)

    n_main = n_chunks // nbuf * nbuf

    def body(p, carry):
        i0 = p * nbuf
        for b in range(nbuf):
            i = i0 + b
            _wait(b)

            @pl.when(i + nbuf - 1 < n_chunks)
            def _():
                _issue(i + nbuf - 1, (b + nbuf - 1) % nbuf)

            _scatter(i, b)
        return carry

    lax.fori_loop(0, n_main // nbuf, body, 0)

    for i in range(n_main, n_chunks):
        b = i % nbuf
        _wait(b)

        @pl.when(i + nbuf - 1 < n_chunks)
        def _():
            _issue(i + nbuf - 1, (b + nbuf - 1) % nbuf)

        _scatter(i, b)


SUPER = 25
_CHUNKS_E = N_EDGES // (NC * NS) // CHUNK
_CHUNKS_C = N_EDGES // NS // CHUNK
_SUPER_E = _CHUNKS_E // SUPER
_SUPER_C = _CHUNKS_C // SUPER

NBUF = 3

_SC_SCRATCH = (
    [pltpu.VMEM((2, 1, 2, SUPER, CHUNK), jnp.int32)]
    + [pltpu.VMEM((CHUNK, D), jnp.float32)] * NBUF
    + [pltpu.VMEM_SHARED((N_NODES, D), jnp.float32)]
    + [pltpu.SemaphoreType.DMA] * (NBUF + 2)
)


def _super_loop(h, edges, lead, n_super, slab, rows, acc, sems):
    gsems, ssems = sems[:NBUF], sems[NBUF:]

    def _load(j, b):
        pltpu.async_copy(edges.at[pl.ds(lead, 1), pl.ds(j, 1)],
                         slab.at[pl.ds(b, 1)], ssems[b])

    def _wait(b):
        pltpu.make_async_copy(edges.at[pl.ds(lead, 1), pl.ds(0, 1)],
                              slab.at[pl.ds(b, 1)], ssems[b]).wait()

    for j in range(n_super):
        b = j & 1
        _wait(b)
        if j + 1 < n_super:
            _load(j + 1, 1 - b)
        _edge_loop(h, slab.at[b, 0, 0], slab.at[b, 0, 1], rows, acc,
                   gsems, SUPER)


_mesh = plsc.VectorSubcoreMesh(core_axis_name="c", subcore_axis_name="s")


@functools.partial(
    pl.kernel,
    out_type=[
        jax.ShapeDtypeStruct((N_NODES, D), jnp.float32),
        jax.ShapeDtypeStruct((N_NODES, D), jnp.float32),
    ],
    mesh=_mesh,
    scratch_types=_SC_SCRATCH,
)
def _prop_edges(h, edges, zeros, p0, p1, slab, *rest):
    rows, acc, sems = rest[:NBUF], rest[NBUF], rest[NBUF + 1:]
    c = lax.axis_index("c")
    s = lax.axis_index("s")
    w = c * NS + s
    pltpu.async_copy(edges.at[pl.ds(w, 1), pl.ds(0, 1)],
                     slab.at[pl.ds(0, 1)], sems[NBUF])
    _zero_acc(zeros, acc, s)
    plsc.subcore_barrier()

    _super_loop(h, edges, w, _SUPER_E, slab, rows, acc, sems)
    plsc.subcore_barrier()

    @pl.when(c == 0)
    def _():
        _copy_out(acc, p0, s)

    @pl.when(c == 1)
    def _():
        _copy_out(acc, p1, s)


@functools.partial(
    pl.kernel,
    out_type=[
        jax.ShapeDtypeStruct((N_NODES, D), jnp.float32),
        jax.ShapeDtypeStruct((N_NODES, D), jnp.float32),
    ],
    mesh=_mesh,
    scratch_types=_SC_SCRATCH,
)
def _prop_cols(h0, h1, edges, zeros, o0, o1, slab, *rest):
    rows, acc, sems = rest[:NBUF], rest[NBUF], rest[NBUF + 1:]
    c = lax.axis_index("c")
    s = lax.axis_index("s")
    pltpu.async_copy(edges.at[pl.ds(s, 1), pl.ds(0, 1)],
                     slab.at[pl.ds(0, 1)], sems[NBUF])
    _zero_acc(zeros, acc, s)
    plsc.subcore_barrier()

    @pl.when(c == 0)
    def _():
        _super_loop(h0, edges, s, _SUPER_C, slab, rows, acc, sems)

    @pl.when(c == 1)
    def _():
        _super_loop(h1, edges, s, _SUPER_C, slab, rows, acc, sems)

    plsc.subcore_barrier()

    @pl.when(c == 0)
    def _():
        _copy_out(acc, o0, s)

    @pl.when(c == 1)
    def _():
        _copy_out(acc, o1, s)


_BLK = 1000
_GRID = N_NODES // _BLK


def _tc1_body(p0, p1, w, b, o0, o1):
    a = p0[...] + p1[...]
    h = jnp.dot(a, w[...], preferred_element_type=jnp.float32) + b[...]
    h = jnp.maximum(h, 0.0)
    o0[...] = h[:, 0:128]
    o1[...] = h[:, 128:256]


def _tc1(p0, p1, w1, b1):
    return pl.pallas_call(
        _tc1_body,
        grid=(_GRID,),
        in_specs=[
            pl.BlockSpec((_BLK, 128), lambda i: (i, 0)),
            pl.BlockSpec((_BLK, 128), lambda i: (i, 0)),
            pl.BlockSpec((128, 256), lambda i: (0, 0)),
            pl.BlockSpec((1, 256), lambda i: (0, 0)),
        ],
        out_specs=[
            pl.BlockSpec((_BLK, 128), lambda i: (i, 0)),
            pl.BlockSpec((_BLK, 128), lambda i: (i, 0)),
        ],
        out_shape=[
            jax.ShapeDtypeStruct((N_NODES, 128), jnp.float32),
            jax.ShapeDtypeStruct((N_NODES, 128), jnp.float32),
        ],
    )(p0, p1, w1, b1)


def _tc2_body(g0, g1, w2, b2, w3, t):
    h = (jnp.dot(g0[...], w2[0:128, :], preferred_element_type=jnp.float32)
         + jnp.dot(g1[...], w2[128:256, :], preferred_element_type=jnp.float32)
         + b2[...])
    h = jnp.maximum(h, 0.0)
    t[...] = jnp.dot(h, w3[...], preferred_element_type=jnp.float32)


def _tc2(g0, g1, w2, b2, w3):
    return pl.pallas_call(
        _tc2_body,
        grid=(_GRID,),
        in_specs=[
            pl.BlockSpec((_BLK, 128), lambda i: (i, 0)),
            pl.BlockSpec((_BLK, 128), lambda i: (i, 0)),
            pl.BlockSpec((256, 256), lambda i: (0, 0)),
            pl.BlockSpec((1, 256), lambda i: (0, 0)),
            pl.BlockSpec((256, 128), lambda i: (0, 0)),
        ],
        out_specs=pl.BlockSpec((_BLK, 128), lambda i: (i, 0)),
        out_shape=jax.ShapeDtypeStruct((N_NODES, 128), jnp.float32),
    )(g0, g1, w2, b2, w3)


def _tc3_body(q0, q1, b3, out):
    out[...] = q0[...] + q1[...] + b3[...]


def _tc3(q0, q1, b3):
    return pl.pallas_call(
        _tc3_body,
        grid=(_GRID,),
        in_specs=[
            pl.BlockSpec((_BLK, 128), lambda i: (i, 0)),
            pl.BlockSpec((_BLK, 128), lambda i: (i, 0)),
            pl.BlockSpec((1, 128), lambda i: (0, 0)),
        ],
        out_specs=pl.BlockSpec((_BLK, 128), lambda i: (i, 0)),
        out_shape=jax.ShapeDtypeStruct((N_NODES, 128), jnp.float32),
    )(q0, q1, b3)


def kernel(x, adj, W1, b1, W2, b2, W3, b3):
    src = adj[0].astype(jnp.int32)
    dst = adj[1].astype(jnp.int32)
    edges_e = jnp.stack(
        [src.reshape(NC * NS, _SUPER_E, SUPER, CHUNK),
         dst.reshape(NC * NS, _SUPER_E, SUPER, CHUNK)], axis=2)
    edges_c = jnp.stack(
        [src.reshape(NS, _SUPER_C, SUPER, CHUNK),
         dst.reshape(NS, _SUPER_C, SUPER, CHUNK)], axis=2)
    zeros = jnp.zeros((ROWS_PER_SUB, D), jnp.float32)

    p0, p1 = _prop_edges(x, edges_e, zeros)
    h0, h1 = _tc1(p0, p1, W1, b1.reshape(1, -1))

    g0, g1 = _prop_cols(h0, h1, edges_c, zeros)
    t = _tc2(g0, g1, W2, b2.reshape(1, -1), W3)

    q0, q1 = _prop_edges(t, edges_e, zeros)
    return _tc3(q0, q1, b3.reshape(1, -1))


# speedup vs baseline: 13.6883x; 1.0849x over previous
import functools

import jax
import jax.numpy as jnp
from jax import lax
from jax.experimental import pallas as pl
from jax.experimental.pallas import tpu as pltpu
from jax.experimental.pallas import tpu_sc as plsc

N_NODES = 10000
N_EDGES = 320000
NS = 16
NC = 2
CHUNK = 80
ROWS_PER_SUB = 624
ROWS_TAIL = N_NODES - NS * ROWS_PER_SUB
D = 128


def _zero_acc(zeros, acc, s):
    row0 = pl.multiple_of(s * ROWS_PER_SUB, 8)
    pltpu.sync_copy(zeros.at[pl.ds(0, ROWS_PER_SUB)],
                    acc.at[pl.ds(row0, ROWS_PER_SUB)])

    @pl.when(s == NS - 1)
    def _():
        pltpu.sync_copy(zeros.at[pl.ds(0, ROWS_TAIL)],
                        acc.at[pl.ds(NS * ROWS_PER_SUB, ROWS_TAIL)])


def _copy_out(acc, out, s):
    row0 = pl.multiple_of(s * ROWS_PER_SUB, 8)
    pltpu.sync_copy(acc.at[pl.ds(row0, ROWS_PER_SUB)],
                    out.at[pl.ds(row0, ROWS_PER_SUB)])

    @pl.when(s == NS - 1)
    def _():
        pltpu.sync_copy(acc.at[pl.ds(NS * ROWS_PER_SUB, ROWS_TAIL)],
                        out.at[pl.ds(NS * ROWS_PER_SUB, ROWS_TAIL)])


def _edge_loop(h, src_slab, dst_slab, rows, acc, sems, n_chunks):
    nbuf = len(rows)

    def _issue(i, b):
        pltpu.async_copy(h.at[src_slab.at[i]], rows[b], sems[b])

    def _wait(b):
        pltpu.make_async_copy(h.at[src_slab.at[0]], rows[b], sems[b]).wait()

    def _scatter(i, b):
        pltpu.sync_copy(rows[b], acc.at[dst_slab.at[i]], add=True)

    for b in range(nbuf - 1):
        _issue(b, b)

    n_main = n_chunks // nbuf * nbuf

    def body(p, carry):
        i0 = p * nbuf
        for b in range(nbuf):
            i = i0 + b
            _wait(b)

            @pl.when(i + nbuf - 1 < n_chunks)
            def _():
                _issue(i + nbuf - 1, (b + nbuf - 1) % nbuf)

            _scatter(i, b)
        return carry

    lax.fori_loop(0, n_main // nbuf, body, 0)

    for i in range(n_main, n_chunks):
        b = i % nbuf
        _wait(b)

        @pl.when(i + nbuf - 1 < n_chunks)
        def _():
            _issue(i + nbuf - 1, (b + nbuf - 1) % nbuf)

        _scatter(i, b)


SUPER = 25
_CHUNKS_E = N_EDGES // (NC * NS) // CHUNK
_CHUNKS_C = N_EDGES // NS // CHUNK
_SUPER_E = _CHUNKS_E // SUPER
_SUPER_C = _CHUNKS_C // SUPER

NBUF = 3

_SC_SCRATCH = (
    [pltpu.VMEM((2, 1, 2, SUPER, CHUNK), jnp.int32)]
    + [pltpu.VMEM((CHUNK, D), jnp.float32)] * NBUF
    + [pltpu.VMEM_SHARED((N_NODES, D), jnp.float32)]
    + [pltpu.SemaphoreType.DMA] * (NBUF + 2)
)


def _super_loop(h, edges, lead, n_super, slab, rows, acc, sems):
    gsems, ssems = sems[:NBUF], sems[NBUF:]
    n_total = n_super * SUPER

    def _load(j):
        pltpu.async_copy(edges.at[pl.ds(lead, 1), pl.ds(j, 1)],
                         slab.at[pl.ds(j & 1, 1)], ssems[j & 1])

    def _wait_slab(j):
        pltpu.make_async_copy(edges.at[pl.ds(lead, 1), pl.ds(0, 1)],
                              slab.at[pl.ds(j & 1, 1)], ssems[j & 1]).wait()

    def _issue(ic):
        pltpu.async_copy(h.at[slab.at[(ic // SUPER) & 1, 0, 0, ic % SUPER]],
                         rows[ic % NBUF], gsems[ic % NBUF])

    def _wait_gather(i):
        pltpu.make_async_copy(h.at[slab.at[0, 0, 0, 0]], rows[i % NBUF],
                              gsems[i % NBUF]).wait()

    def _scatter(i):
        pltpu.sync_copy(rows[i % NBUF],
                        acc.at[slab.at[(i // SUPER) & 1, 0, 1, i % SUPER]],
                        add=True)

    _wait_slab(0)
    if n_super > 1:
        _load(1)
    for b in range(NBUF - 1):
        _issue(b)

    for i in range(n_total):
        ic = i + NBUF - 1
        if ic < n_total and ic % SUPER == 0:
            _wait_slab(ic // SUPER)
        if i % SUPER == 0 and i > 0 and i // SUPER + 1 < n_super:
            _load(i // SUPER + 1)
        if ic < n_total:
            _issue(ic)
        _wait_gather(i)
        _scatter(i)


_mesh = plsc.VectorSubcoreMesh(core_axis_name="c", subcore_axis_name="s")


@functools.partial(
    pl.kernel,
    out_type=[
        jax.ShapeDtypeStruct((N_NODES, D), jnp.float32),
        jax.ShapeDtypeStruct((N_NODES, D), jnp.float32),
    ],
    mesh=_mesh,
    scratch_types=_SC_SCRATCH,
)
def _prop_edges(h, edges, zeros, p0, p1, slab, *rest):
    rows, acc, sems = rest[:NBUF], rest[NBUF], rest[NBUF + 1:]
    c = lax.axis_index("c")
    s = lax.axis_index("s")
    w = c * NS + s
    pltpu.async_copy(edges.at[pl.ds(w, 1), pl.ds(0, 1)],
                     slab.at[pl.ds(0, 1)], sems[NBUF])
    _zero_acc(zeros, acc, s)
    plsc.subcore_barrier()

    _super_loop(h, edges, w, _SUPER_E, slab, rows, acc, sems)
    plsc.subcore_barrier()

    @pl.when(c == 0)
    def _():
        _copy_out(acc, p0, s)

    @pl.when(c == 1)
    def _():
        _copy_out(acc, p1, s)


@functools.partial(
    pl.kernel,
    out_type=[
        jax.ShapeDtypeStruct((N_NODES, D), jnp.float32),
        jax.ShapeDtypeStruct((N_NODES, D), jnp.float32),
    ],
    mesh=_mesh,
    scratch_types=_SC_SCRATCH,
)
def _prop_cols(h0, h1, edges, zeros, o0, o1, slab, *rest):
    rows, acc, sems = rest[:NBUF], rest[NBUF], rest[NBUF + 1:]
    c = lax.axis_index("c")
    s = lax.axis_index("s")
    pltpu.async_copy(edges.at[pl.ds(s, 1), pl.ds(0, 1)],
                     slab.at[pl.ds(0, 1)], sems[NBUF])
    _zero_acc(zeros, acc, s)
    plsc.subcore_barrier()

    @pl.when(c == 0)
    def _():
        _super_loop(h0, edges, s, _SUPER_C, slab, rows, acc, sems)

    @pl.when(c == 1)
    def _():
        _super_loop(h1, edges, s, _SUPER_C, slab, rows, acc, sems)

    plsc.subcore_barrier()

    @pl.when(c == 0)
    def _():
        _copy_out(acc, o0, s)

    @pl.when(c == 1)
    def _():
        _copy_out(acc, o1, s)


_BLK = 2000
_GRID = N_NODES // _BLK


def _tc1_body(p0, p1, w, b, o0, o1):
    a = p0[...] + p1[...]
    h = jnp.dot(a, w[...], preferred_element_type=jnp.float32) + b[...]
    h = jnp.maximum(h, 0.0)
    o0[...] = h[:, 0:128]
    o1[...] = h[:, 128:256]


def _tc1(p0, p1, w1, b1):
    return pl.pallas_call(
        _tc1_body,
        grid=(_GRID,),
        in_specs=[
            pl.BlockSpec((_BLK, 128), lambda i: (i, 0)),
            pl.BlockSpec((_BLK, 128), lambda i: (i, 0)),
            pl.BlockSpec((128, 256), lambda i: (0, 0)),
            pl.BlockSpec((1, 256), lambda i: (0, 0)),
        ],
        out_specs=[
            pl.BlockSpec((_BLK, 128), lambda i: (i, 0)),
            pl.BlockSpec((_BLK, 128), lambda i: (i, 0)),
        ],
        out_shape=[
            jax.ShapeDtypeStruct((N_NODES, 128), jnp.float32),
            jax.ShapeDtypeStruct((N_NODES, 128), jnp.float32),
        ],
    )(p0, p1, w1, b1)


def _tc2_body(g0, g1, w2, b2, w3, t):
    h = (jnp.dot(g0[...], w2[0:128, :], preferred_element_type=jnp.float32)
         + jnp.dot(g1[...], w2[128:256, :], preferred_element_type=jnp.float32)
         + b2[...])
    h = jnp.maximum(h, 0.0)
    t[...] = jnp.dot(h, w3[...], preferred_element_type=jnp.float32)


def _tc2(g0, g1, w2, b2, w3):
    return pl.pallas_call(
        _tc2_body,
        grid=(_GRID,),
        in_specs=[
            pl.BlockSpec((_BLK, 128), lambda i: (i, 0)),
            pl.BlockSpec((_BLK, 128), lambda i: (i, 0)),
            pl.BlockSpec((256, 256), lambda i: (0, 0)),
            pl.BlockSpec((1, 256), lambda i: (0, 0)),
            pl.BlockSpec((256, 128), lambda i: (0, 0)),
        ],
        out_specs=pl.BlockSpec((_BLK, 128), lambda i: (i, 0)),
        out_shape=jax.ShapeDtypeStruct((N_NODES, 128), jnp.float32),
    )(g0, g1, w2, b2, w3)


def _tc3_body(q0, q1, b3, out):
    out[...] = q0[...] + q1[...] + b3[...]


def _tc3(q0, q1, b3):
    return pl.pallas_call(
        _tc3_body,
        grid=(_GRID,),
        in_specs=[
            pl.BlockSpec((_BLK, 128), lambda i: (i, 0)),
            pl.BlockSpec((_BLK, 128), lambda i: (i, 0)),
            pl.BlockSpec((1, 128), lambda i: (0, 0)),
        ],
        out_specs=pl.BlockSpec((_BLK, 128), lambda i: (i, 0)),
        out_shape=jax.ShapeDtypeStruct((N_NODES, 128), jnp.float32),
    )(q0, q1, b3)


def kernel(x, adj, W1, b1, W2, b2, W3, b3):
    src = adj[0].astype(jnp.int32)
    dst = adj[1].astype(jnp.int32)
    edges_e = jnp.stack(
        [src.reshape(NC * NS, _SUPER_E, SUPER, CHUNK),
         dst.reshape(NC * NS, _SUPER_E, SUPER, CHUNK)], axis=2)
    edges_c = jnp.stack(
        [src.reshape(NS, _SUPER_C, SUPER, CHUNK),
         dst.reshape(NS, _SUPER_C, SUPER, CHUNK)], axis=2)
    zeros = jnp.zeros((ROWS_PER_SUB, D), jnp.float32)

    p0, p1 = _prop_edges(x, edges_e, zeros)
    h0, h1 = _tc1(p0, p1, W1, b1.reshape(1, -1))

    g0, g1 = _prop_cols(h0, h1, edges_c, zeros)
    t = _tc2(g0, g1, W2, b2.reshape(1, -1), W3)

    q0, q1 = _prop_edges(t, edges_e, zeros)
    return _tc3(q0, q1, b3.reshape(1, -1))


# speedup vs baseline: 13.8838x; 1.0143x over previous
import functools

import jax
import jax.numpy as jnp
from jax import lax
from jax.experimental import pallas as pl
from jax.experimental.pallas import tpu as pltpu
from jax.experimental.pallas import tpu_sc as plsc

N_NODES = 10000
N_EDGES = 320000
NS = 16
NC = 2
CHUNK = 80
ROWS_PER_SUB = 624
ROWS_TAIL = N_NODES - NS * ROWS_PER_SUB
D = 128


def _zero_acc(zeros, acc, s):
    row0 = pl.multiple_of(s * ROWS_PER_SUB, 8)
    pltpu.sync_copy(zeros.at[pl.ds(0, ROWS_PER_SUB)],
                    acc.at[pl.ds(row0, ROWS_PER_SUB)])

    @pl.when(s == NS - 1)
    def _():
        pltpu.sync_copy(zeros.at[pl.ds(0, ROWS_TAIL)],
                        acc.at[pl.ds(NS * ROWS_PER_SUB, ROWS_TAIL)])


def _copy_out(acc, out, s):
    row0 = pl.multiple_of(s * ROWS_PER_SUB, 8)
    pltpu.sync_copy(acc.at[pl.ds(row0, ROWS_PER_SUB)],
                    out.at[pl.ds(row0, ROWS_PER_SUB)])

    @pl.when(s == NS - 1)
    def _():
        pltpu.sync_copy(acc.at[pl.ds(NS * ROWS_PER_SUB, ROWS_TAIL)],
                        out.at[pl.ds(NS * ROWS_PER_SUB, ROWS_TAIL)])


def _edge_loop(h, src_slab, dst_slab, rows, acc, sems, n_chunks):
    nbuf = len(rows)

    def _issue(i, b):
        pltpu.async_copy(h.at[src_slab.at[i]], rows[b], sems[b])

    def _wait(b):
        pltpu.make_async_copy(h.at[src_slab.at[0]], rows[b], sems[b]).wait()

    def _scatter(i, b):
        pltpu.sync_copy(rows[b], acc.at[dst_slab.at[i]], add=True)

    for b in range(nbuf - 1):
        _issue(b, b)

    n_main = n_chunks // nbuf * nbuf

    def body(p, carry):
        i0 = p * nbuf
        for b in range(nbuf):
            i = i0 + b
            _wait(b)

            @pl.when(i + nbuf - 1 < n_chunks)
            def _():
                _issue(i + nbuf - 1, (b + nbuf - 1) % nbuf)

            _scatter(i, b)
        return carry

    lax.fori_loop(0, n_main // nbuf, body, 0)

    for i in range(n_main, n_chunks):
        b = i % nbuf
        _wait(b)

        @pl.when(i + nbuf - 1 < n_chunks)
        def _():
            _issue(i + nbuf - 1, (b + nbuf - 1) % nbuf)

        _scatter(i, b)


SUPER = 25
_CHUNKS_E = N_EDGES // (NC * NS) // CHUNK
_CHUNKS_C = N_EDGES // NS // CHUNK
_SUPER_E = _CHUNKS_E // SUPER
_SUPER_C = _CHUNKS_C // SUPER

NBUF = 3

_SC_SCRATCH = (
    [pltpu.VMEM((2, 1, SUPER, CHUNK), jnp.int32)] * 2
    + [pltpu.VMEM((CHUNK, D), jnp.float32)] * NBUF
    + [pltpu.VMEM_SHARED((N_NODES, D), jnp.float32)]
    + [pltpu.SemaphoreType.DMA] * (NBUF + 2)
)


def _super_loop(h, src, dst, lead, n_super, src_slab, dst_slab,
                rows, acc, sems):
    gsems, ssems = sems[:NBUF], sems[NBUF:]
    n_total = n_super * SUPER

    def _load(j):
        pltpu.async_copy(src.at[pl.ds(lead, 1), pl.ds(j, 1)],
                         src_slab.at[pl.ds(j & 1, 1)], ssems[j & 1])
        pltpu.async_copy(dst.at[pl.ds(lead, 1), pl.ds(j, 1)],
                         dst_slab.at[pl.ds(j & 1, 1)], ssems[j & 1])

    def _wait_slab(j):
        pltpu.make_async_copy(src.at[pl.ds(lead, 1), pl.ds(0, 1)],
                              src_slab.at[pl.ds(j & 1, 1)],
                              ssems[j & 1]).wait()
        pltpu.make_async_copy(dst.at[pl.ds(lead, 1), pl.ds(0, 1)],
                              dst_slab.at[pl.ds(j & 1, 1)],
                              ssems[j & 1]).wait()

    def _issue(ic):
        pltpu.async_copy(h.at[src_slab.at[(ic // SUPER) & 1, 0, ic % SUPER]],
                         rows[ic % NBUF], gsems[ic % NBUF])

    def _wait_gather(i):
        pltpu.make_async_copy(h.at[src_slab.at[0, 0, 0]], rows[i % NBUF],
                              gsems[i % NBUF]).wait()

    def _scatter(i):
        pltpu.sync_copy(rows[i % NBUF],
                        acc.at[dst_slab.at[(i // SUPER) & 1, 0, i % SUPER]],
                        add=True)

    _wait_slab(0)
    if n_super > 1:
        _load(1)
    for b in range(NBUF - 1):
        _issue(b)

    for i in range(n_total):
        ic = i + NBUF - 1
        if ic < n_total and ic % SUPER == 0:
            _wait_slab(ic // SUPER)
        if i % SUPER == 0 and i > 0 and i // SUPER + 1 < n_super:
            _load(i // SUPER + 1)
        if ic < n_total:
            _issue(ic)
        _wait_gather(i)
        _scatter(i)


_mesh = plsc.VectorSubcoreMesh(core_axis_name="c", subcore_axis_name="s")


@functools.partial(
    pl.kernel,
    out_type=[
        jax.ShapeDtypeStruct((N_NODES, D), jnp.float32),
        jax.ShapeDtypeStruct((N_NODES, D), jnp.float32),
    ],
    mesh=_mesh,
    scratch_types=_SC_SCRATCH,
)
def _prop_edges(h, src, dst, zeros, p0, p1, src_slab, dst_slab, *rest):
    rows, acc, sems = rest[:NBUF], rest[NBUF], rest[NBUF + 1:]
    c = lax.axis_index("c")
    s = lax.axis_index("s")
    w = c * NS + s
    pltpu.async_copy(src.at[pl.ds(w, 1), pl.ds(0, 1)],
                     src_slab.at[pl.ds(0, 1)], sems[NBUF])
    pltpu.async_copy(dst.at[pl.ds(w, 1), pl.ds(0, 1)],
                     dst_slab.at[pl.ds(0, 1)], sems[NBUF])
    _zero_acc(zeros, acc, s)
    plsc.subcore_barrier()

    _super_loop(h, src, dst, w, _SUPER_E, src_slab, dst_slab, rows, acc, sems)
    plsc.subcore_barrier()

    @pl.when(c == 0)
    def _():
        _copy_out(acc, p0, s)

    @pl.when(c == 1)
    def _():
        _copy_out(acc, p1, s)


@functools.partial(
    pl.kernel,
    out_type=[
        jax.ShapeDtypeStruct((N_NODES, D), jnp.float32),
        jax.ShapeDtypeStruct((N_NODES, D), jnp.float32),
    ],
    mesh=_mesh,
    scratch_types=_SC_SCRATCH,
)
def _prop_cols(h0, h1, src, dst, zeros, o0, o1, src_slab, dst_slab, *rest):
    rows, acc, sems = rest[:NBUF], rest[NBUF], rest[NBUF + 1:]
    c = lax.axis_index("c")
    s = lax.axis_index("s")
    pltpu.async_copy(src.at[pl.ds(s, 1), pl.ds(0, 1)],
                     src_slab.at[pl.ds(0, 1)], sems[NBUF])
    pltpu.async_copy(dst.at[pl.ds(s, 1), pl.ds(0, 1)],
                     dst_slab.at[pl.ds(0, 1)], sems[NBUF])
    _zero_acc(zeros, acc, s)
    plsc.subcore_barrier()

    @pl.when(c == 0)
    def _():
        _super_loop(h0, src, dst, s, _SUPER_C, src_slab, dst_slab,
                    rows, acc, sems)

    @pl.when(c == 1)
    def _():
        _super_loop(h1, src, dst, s, _SUPER_C, src_slab, dst_slab,
                    rows, acc, sems)

    plsc.subcore_barrier()

    @pl.when(c == 0)
    def _():
        _copy_out(acc, o0, s)

    @pl.when(c == 1)
    def _():
        _copy_out(acc, o1, s)


_BLK = 2000
_GRID = N_NODES // _BLK


def _tc1_body(p0, p1, w, b, o0, o1):
    a = p0[...] + p1[...]
    h = jnp.dot(a, w[...], preferred_element_type=jnp.float32) + b[...]
    h = jnp.maximum(h, 0.0)
    o0[...] = h[:, 0:128]
    o1[...] = h[:, 128:256]


def _tc1(p0, p1, w1, b1):
    return pl.pallas_call(
        _tc1_body,
        grid=(_GRID,),
        in_specs=[
            pl.BlockSpec((_BLK, 128), lambda i: (i, 0)),
            pl.BlockSpec((_BLK, 128), lambda i: (i, 0)),
            pl.BlockSpec((128, 256), lambda i: (0, 0)),
            pl.BlockSpec((1, 256), lambda i: (0, 0)),
        ],
        out_specs=[
            pl.BlockSpec((_BLK, 128), lambda i: (i, 0)),
            pl.BlockSpec((_BLK, 128), lambda i: (i, 0)),
        ],
        out_shape=[
            jax.ShapeDtypeStruct((N_NODES, 128), jnp.float32),
            jax.ShapeDtypeStruct((N_NODES, 128), jnp.float32),
        ],
    )(p0, p1, w1, b1)


def _tc2_body(g0, g1, w2, b2, w3, t):
    h = (jnp.dot(g0[...], w2[0:128, :], preferred_element_type=jnp.float32)
         + jnp.dot(g1[...], w2[128:256, :], preferred_element_type=jnp.float32)
         + b2[...])
    h = jnp.maximum(h, 0.0)
    t[...] = jnp.dot(h, w3[...], preferred_element_type=jnp.float32)


def _tc2(g0, g1, w2, b2, w3):
    return pl.pallas_call(
        _tc2_body,
        grid=(_GRID,),
        in_specs=[
            pl.BlockSpec((_BLK, 128), lambda i: (i, 0)),
            pl.BlockSpec((_BLK, 128), lambda i: (i, 0)),
            pl.BlockSpec((256, 256), lambda i: (0, 0)),
            pl.BlockSpec((1, 256), lambda i: (0, 0)),
            pl.BlockSpec((256, 128), lambda i: (0, 0)),
        ],
        out_specs=pl.BlockSpec((_BLK, 128), lambda i: (i, 0)),
        out_shape=jax.ShapeDtypeStruct((N_NODES, 128), jnp.float32),
    )(g0, g1, w2, b2, w3)


def _tc3_body(q0, q1, b3, out):
    out[...] = q0[...] + q1[...] + b3[...]


def _tc3(q0, q1, b3):
    return pl.pallas_call(
        _tc3_body,
        grid=(_GRID,),
        in_specs=[
            pl.BlockSpec((_BLK, 128), lambda i: (i, 0)),
            pl.BlockSpec((_BLK, 128), lambda i: (i, 0)),
            pl.BlockSpec((1, 128), lambda i: (0, 0)),
        ],
        out_specs=pl.BlockSpec((_BLK, 128), lambda i: (i, 0)),
        out_shape=jax.ShapeDtypeStruct((N_NODES, 128), jnp.float32),
    )(q0, q1, b3)


def kernel(x, adj, W1, b1, W2, b2, W3, b3):
    src = adj[0].astype(jnp.int32)
    dst = adj[1].astype(jnp.int32)
    src_e = src.reshape(NC * NS, _SUPER_E, SUPER, CHUNK)
    dst_e = dst.reshape(NC * NS, _SUPER_E, SUPER, CHUNK)
    src_c = src.reshape(NS, _SUPER_C, SUPER, CHUNK)
    dst_c = dst.reshape(NS, _SUPER_C, SUPER, CHUNK)
    zeros = jnp.zeros((ROWS_PER_SUB, D), jnp.float32)

    p0, p1 = _prop_edges(x, src_e, dst_e, zeros)
    h0, h1 = _tc1(p0, p1, W1, b1.reshape(1, -1))

    g0, g1 = _prop_cols(h0, h1, src_c, dst_c, zeros)
    t = _tc2(g0, g1, W2, b2.reshape(1, -1), W3)

    q0, q1 = _prop_edges(t, src_e, dst_e, zeros)
    return _tc3(q0, q1, b3.reshape(1, -1))
